# baseline (speedup 1.0000x reference)
import dataclasses
import functools

import jax
import jax.numpy as jnp
from jax import lax
from jax.experimental import pallas as pl
from jax.experimental.pallas import tpu as pltpu
from jax.experimental.pallas import tpu_sc as plsc

N = 10000
NP = 10240
E = 320000
GAMMA = 0.1
K_ITERS = 6
D = 128

NTILES = 16
LANES = 16
PIECES = D // LANES
BATCH = 1024
EP = 327680
NW = 32
RNG = NP // NW
ZR = 10112
ZSTRIPE = ZR // NTILES

_MESH = plsc.VectorSubcoreMesh(core_axis_name="core", subcore_axis_name="subcore")

_GDN = lax.GatherDimensionNumbers(
    offset_dims=(), collapsed_slice_dims=(0,), start_index_map=(0,))


def _dense_body(act, has_skip, x_ref, wt_ref, b_ref, *rest):
    if has_skip:
        z_ref, o_ref = rest
    else:
        (o_ref,) = rest
    o = jax.lax.dot_general(
        x_ref[...], wt_ref[...], (((1,), (0,)), ((), ())),
        preferred_element_type=jnp.float32,
        precision=jax.lax.Precision.HIGHEST)
    o = o + b_ref[...]
    if has_skip:
        o = o + z_ref[...]
    if act == "elu":
        o = jnp.where(o > 0, o, jnp.exp(jnp.minimum(o, 0.0)) - 1.0)
    o_ref[...] = o


def _dense(x, W, b, z=None, act="none", block=1024):
    n, din = x.shape
    dout = W.shape[0]
    wt = W.T
    b2 = b.reshape(1, dout)
    in_specs = [
        pl.BlockSpec((block, din), lambda i: (i, 0)),
        pl.BlockSpec((din, dout), lambda i: (0, 0)),
        pl.BlockSpec((1, dout), lambda i: (0, 0)),
    ]
    args = [x, wt, b2]
    if z is not None:
        in_specs.append(pl.BlockSpec((block, dout), lambda i: (i, 0)))
        args.append(z)
    return pl.pallas_call(
        functools.partial(_dense_body, act, z is not None),
        grid=(n // block,),
        in_specs=in_specs,
        out_specs=pl.BlockSpec((block, dout), lambda i: (i, 0)),
        out_shape=jax.ShapeDtypeStruct((n, dout), jnp.float32),
    )(*args)


def _bcast16(vec16, ii):
    return lax.gather(vec16, jnp.full((LANES, 1), ii, jnp.int32),
                      _GDN, (1,), mode=lax.GatherScatterMode.PROMISE_IN_BOUNDS)


def _extract(vec_ref, c, t):
    v = vec_ref[pl.ds(c * LANES, LANES)]
    lane = lax.iota(jnp.int32, LANES)
    return jax.lax.reduce_sum_p.bind(
        jnp.where(lane == t, v, 0), axes=(0,))


def _ppr_step_body(z_hbm, xp_hbm, src_hbm, dst_hbm, w_hbm, lo_hbm, hi_hbm,
                   out_hbm,
                   z_sh, acc_v, rows0_v, rows1_v, sidx_v, didx_v, wv_v,
                   lov_v, hiv_v, sem_i, sem_g0, sem_g1):
    c = lax.axis_index("core")
    t = lax.axis_index("subcore")
    w = c * NTILES + t

    cols = [lax.iota(jnp.int32, LANES) + p * LANES for p in range(PIECES)]
    lane = lax.iota(jnp.int32, LANES)

    pltpu.sync_copy(z_hbm.at[pl.ds(t * ZSTRIPE, ZSTRIPE)],
                    z_sh.at[pl.ds(t * ZSTRIPE, ZSTRIPE)])

    @pl.loop(0, RNG)
    def _(i):
        for p in range(PIECES):
            acc_v[i, pl.ds(p * LANES, LANES)] = jnp.zeros((LANES,), jnp.float32)

    pltpu.sync_copy(lo_hbm, lov_v)
    pltpu.sync_copy(hi_hbm, hiv_v)
    lo = _extract(lov_v, c, t)
    hi = _extract(hiv_v, c, t)
    base = w * RNG

    plsc.subcore_barrier()

    @pl.loop(lo // BATCH, (hi + BATCH - 1) // BATCH)
    def _(bb):
        e0 = bb * BATCH
        hi_ = pltpu.async_copy(src_hbm.at[pl.ds(e0, BATCH)], sidx_v, sem_i)
        hd_ = pltpu.async_copy(dst_hbm.at[pl.ds(e0, BATCH)], didx_v, sem_i)
        hw_ = pltpu.async_copy(w_hbm.at[pl.ds(e0, BATCH)], wv_v, sem_i)
        hi_.wait()
        hd_.wait()
        hw_.wait()

        def sub(s, rb, sem):
            return pltpu.async_copy(
                z_sh.at[sidx_v.at[pl.ds(s * LANES, LANES)]], rb, sem)

        def compute(s, rb):
            go = s * LANES
            eid = e0 + go + lane
            valid = (eid >= lo) & (eid < hi)
            w16 = jnp.where(valid, wv_v[pl.ds(go, LANES)], 0.0)
            d16 = jnp.clip(didx_v[pl.ds(go, LANES)] - base, 0, RNG - 1)

            @pl.loop(0, LANES, unroll=4)
            def _(ii):
                wb = _bcast16(w16, ii)
                db = _bcast16(d16, ii)
                for p in range(PIECES):
                    v = rb[ii, pl.ds(p * LANES, LANES)] * wb
                    plsc.addupdate_scatter(acc_v, [db, cols[p]], v)

        @pl.loop(0, BATCH // LANES, step=2)
        def _(ss):
            h0 = sub(ss, rows0_v, sem_g0)
            h1 = sub(ss + 1, rows1_v, sem_g1)
            h0.wait()
            compute(ss, rows0_v)
            h1.wait()
            compute(ss + 1, rows1_v)

    @pl.loop(0, RNG // LANES)
    def _(u):
        r0 = base + u * LANES
        pltpu.sync_copy(xp_hbm.at[pl.ds(r0, LANES)], rows0_v)

        @pl.loop(0, LANES)
        def _(i):
            for p in range(PIECES):
                sl = pl.ds(p * LANES, LANES)
                v = GAMMA * acc_v[u * LANES + i, sl] + rows0_v[i, sl]
                rows1_v[i, sl] = jnp.maximum(v, 0.0)

        pltpu.sync_copy(rows1_v, out_hbm.at[pl.ds(r0, LANES)])


def _make_ppr_step():
    cp = pltpu.CompilerParams()
    if "needs_layout_passes" in pltpu.CompilerParams.__dataclass_fields__:
        cp = dataclasses.replace(cp, needs_layout_passes=False)
    return pl.kernel(
        _ppr_step_body,
        out_type=jax.ShapeDtypeStruct((NP, D), jnp.float32),
        mesh=_MESH,
        compiler_params=cp,
        scratch_types=[
            pltpu.VMEM_SHARED((ZR, D), jnp.float32),
            pltpu.VMEM((RNG, D), jnp.float32),
            pltpu.VMEM((LANES, D), jnp.float32),
            pltpu.VMEM((LANES, D), jnp.float32),
            pltpu.VMEM((BATCH,), jnp.int32),
            pltpu.VMEM((BATCH,), jnp.int32),
            pltpu.VMEM((BATCH,), jnp.float32),
            pltpu.VMEM((NW,), jnp.int32),
            pltpu.VMEM((NW,), jnp.int32),
            pltpu.SemaphoreType.DMA,
            pltpu.SemaphoreType.DMA,
            pltpu.SemaphoreType.DMA,
        ],
    )


def kernel(features, edge_index, edge_weight,
           W1, b1, W2, b2, W3, b3, W4, b4, W5, b5,
           VW0, Vb0, VW1, Vb1, VW2, Vb2, VW3, Vb3, VW, Vb):
    dst_s, src_s, w_s = jax.lax.sort(
        (edge_index[0], edge_index[1], edge_weight), num_keys=1)
    pad = EP - E
    dst_p = jnp.concatenate([dst_s, jnp.full((pad,), N - 1, jnp.int32)])
    src_p = jnp.concatenate([src_s, jnp.zeros((pad,), jnp.int32)])
    w_p = jnp.concatenate([w_s, jnp.zeros((pad,), jnp.float32)])
    bounds = jnp.arange(NW, dtype=jnp.int32) * RNG
    lo = jnp.searchsorted(dst_p, bounds, side="left").astype(jnp.int32)
    hi = jnp.concatenate([lo[1:], jnp.array([EP], jnp.int32)])

    step = _make_ppr_step()

    def ppr_slab(xp):
        z = xp
        for _ in range(K_ITERS):
            z = step(z, xp, src_p, dst_p, w_p, lo, hi)
        return z

    def ppr(xp):
        d = xp.shape[1]
        if d == D:
            return ppr_slab(xp)
        return jnp.concatenate(
            [ppr_slab(xp[:, i:i + D]) for i in range(0, d, D)], axis=1)

    x = jnp.pad(features, ((0, NP - N), (0, 0)))
    z = ppr(_dense(x, W1, b1))
    x = _dense(x, VW0, Vb0, z=z, act="elu")
    z = ppr(_dense(x, W2, b2))
    x = _dense(x, VW1, Vb1, z=z, act="elu")
    z = ppr(_dense(x, W3, b3))
    x = _dense(x, VW2, Vb2, z=z, act="elu")
    W4p = jnp.pad(W4, ((0, 64), (0, 0)))
    b4p = jnp.pad(b4, (0, 64))
    z = ppr(_dense(x, W4p, b4p))[:, :64]
    x = _dense(x, VW3, Vb3, z=z, act="elu")
    W5p = jnp.pad(W5, ((0, 7), (0, 0)))
    b5p = jnp.pad(b5, (0, 7))
    VWp = jnp.pad(VW, ((0, 7), (0, 0)))
    Vbp = jnp.pad(Vb, (0, 7))
    z = ppr(_dense(x, W5p, b5p))
    out = _dense(x, VWp, Vbp, z=z)[:N, :121]
    return (out, K_ITERS * 5)
